# baseline (speedup 1.0000x reference)
import functools

import jax
import jax.numpy as jnp
from jax import lax
from jax.experimental import pallas as pl
from jax.experimental.pallas import tpu as pltpu
from jax.experimental.pallas import tpu_sc as plsc

NC = 2
NS = 16
NW = NC * NS
CH = 128


@functools.cache
def _make_prop(n: int, npad: int, d: int, e: int):
    ept = e // NW
    nch = ept // CH
    assert ept * NW == e and nch * CH == ept
    rpt = npad // NS
    assert rpt * NS == npad and rpt % 8 == 0
    mesh = plsc.VectorSubcoreMesh(core_axis_name="c", subcore_axis_name="s")

    @functools.partial(
        pl.kernel,
        out_type=jax.ShapeDtypeStruct((NC, npad, d), jnp.float32),
        mesh=mesh,
        scratch_types=[
            pltpu.VMEM((nch, CH), jnp.int32),
            pltpu.VMEM((nch, CH), jnp.int32),
            pltpu.VMEM((CH, d), jnp.float32),
            pltpu.VMEM_SHARED((npad, d), jnp.float32),
            pltpu.SemaphoreType.DMA,
        ],
    )
    def prop(y_hbm, srcs_hbm, dsts_hbm, zeros_hbm, out_hbm,
             src_v, dst_v, rows_v, accum, sem):
        c = lax.axis_index("c")
        s = lax.axis_index("s")
        wid = c * NS + s
        pltpu.sync_copy(srcs_hbm.at[wid], src_v)
        pltpu.sync_copy(dsts_hbm.at[wid], dst_v)
        r0 = s * rpt
        pltpu.sync_copy(zeros_hbm.at[pl.ds(r0, rpt)], accum.at[pl.ds(r0, rpt)])
        plsc.subcore_barrier()

        def body(j, carry):
            pltpu.async_copy(y_hbm.at[src_v.at[j]], rows_v, sem).wait()
            pltpu.sync_copy(rows_v, accum.at[dst_v.at[j]], add=True)
            return carry

        lax.fori_loop(0, nch, body, 0)
        plsc.subcore_barrier()
        pltpu.sync_copy(accum.at[pl.ds(r0, rpt)], out_hbm.at[c, pl.ds(r0, rpt)])

    return prop


@functools.cache
def _make_deg(n: int, e: int):
    ept = e // NW
    assert ept * NW == e and ept % 16 == 0
    mesh = plsc.VectorSubcoreMesh(core_axis_name="c", subcore_axis_name="s")

    @functools.partial(
        pl.kernel,
        out_type=jax.ShapeDtypeStruct((NW, n), jnp.float32),
        mesh=mesh,
        scratch_types=[
            pltpu.VMEM((ept,), jnp.int32),
            pltpu.VMEM((n,), jnp.float32),
        ],
        compiler_params=pltpu.CompilerParams(needs_layout_passes=False),
    )
    def deg(dsts_hbm, zeros_hbm, out_hbm, dst_v, cnt_v):
        c = lax.axis_index("c")
        s = lax.axis_index("s")
        wid = c * NS + s
        pltpu.sync_copy(dsts_hbm.at[wid], dst_v)
        pltpu.sync_copy(zeros_hbm, cnt_v)
        ones16 = jnp.ones((16,), jnp.float32)

        def body(i, carry):
            idx = dst_v[pl.ds(i * 16, 16)]
            plsc.addupdate_scatter(cnt_v, [idx], ones16)
            return carry

        lax.fori_loop(0, ept // 16, body, 0)
        pltpu.sync_copy(cnt_v, out_hbm.at[wid])

    return deg


def _blk(n):
    return next(b for b in (2048, 2000, 1280, 1000, 640, 512, 400, 200, 8)
                if n % b == 0)


def _pre_body(cnt_ref, x_ref, w_ref, dinv_ref, y_ref):
    deg = 1.0 + jnp.sum(cnt_ref[...], axis=1)
    dinv = lax.rsqrt(deg)[:, None]
    dinv_ref[...] = dinv
    y_ref[...] = jnp.dot(x_ref[...], w_ref[...],
                         preferred_element_type=jnp.float32) * dinv


def _pre(cnt_t, x, w):
    n, din = x.shape
    d = w.shape[1]
    blk = _blk(n)
    return pl.pallas_call(
        _pre_body,
        grid=(n // blk,),
        in_specs=[pl.BlockSpec((blk, NW), lambda i: (i, 0)),
                  pl.BlockSpec((blk, din), lambda i: (i, 0)),
                  pl.BlockSpec((din, d), lambda i: (0, 0))],
        out_specs=[pl.BlockSpec((blk, 1), lambda i: (i, 0)),
                   pl.BlockSpec((blk, d), lambda i: (i, 0))],
        out_shape=[jax.ShapeDtypeStruct((n, 1), jnp.float32),
                   jax.ShapeDtypeStruct((n, d), jnp.float32)],
    )(cnt_t, x, w)


def _mid_body(p_ref, y1_ref, dinv_ref, b1_ref, y2_ref):
    dinv = dinv_ref[...]
    pre = dinv * (p_ref[0] + p_ref[1] + y1_ref[...]) + b1_ref[...][None, :]
    y2_ref[...] = dinv * jnp.maximum(pre, 0.0)


def _mid(p, y1, dinv, b1):
    n, d = y1.shape
    blk = _blk(n)
    return pl.pallas_call(
        _mid_body,
        grid=(n // blk,),
        in_specs=[pl.BlockSpec((NC, blk, d), lambda i: (0, i, 0)),
                  pl.BlockSpec((blk, d), lambda i: (i, 0)),
                  pl.BlockSpec((blk, 1), lambda i: (i, 0)),
                  pl.BlockSpec((d,), lambda i: (0,))],
        out_specs=pl.BlockSpec((blk, d), lambda i: (i, 0)),
        out_shape=jax.ShapeDtypeStruct((n, d), jnp.float32),
    )(p, y1, dinv, b1)


def _final_body(q_ref, y2_ref, dinv_ref, wmu_ref, bmu_ref, wls_ref, bls_ref,
                mu_ref, ls_ref):
    g = dinv_ref[...] * (q_ref[0] + q_ref[1] + y2_ref[...])
    mu_ref[...] = (jnp.dot(g, wmu_ref[...], preferred_element_type=jnp.float32)
                   + bmu_ref[...][None, :])
    ls_ref[...] = (jnp.dot(g, wls_ref[...], preferred_element_type=jnp.float32)
                   + bls_ref[...][None, :])


def _final(q, y2, dinv, wmu, bmu, wls, bls):
    n, d = y2.shape
    dout = wmu.shape[1]
    blk = _blk(n)
    return pl.pallas_call(
        _final_body,
        grid=(n // blk,),
        in_specs=[pl.BlockSpec((NC, blk, d), lambda i: (0, i, 0)),
                  pl.BlockSpec((blk, d), lambda i: (i, 0)),
                  pl.BlockSpec((blk, 1), lambda i: (i, 0)),
                  pl.BlockSpec((d, dout), lambda i: (0, 0)),
                  pl.BlockSpec((dout,), lambda i: (0,)),
                  pl.BlockSpec((d, dout), lambda i: (0, 0)),
                  pl.BlockSpec((dout,), lambda i: (0,))],
        out_specs=[pl.BlockSpec((blk, dout), lambda i: (i, 0)),
                   pl.BlockSpec((blk, dout), lambda i: (i, 0))],
        out_shape=[jax.ShapeDtypeStruct((n, dout), jnp.float32),
                   jax.ShapeDtypeStruct((n, dout), jnp.float32)],
    )(q, y2, dinv, wmu, bmu, wls, bls)


def kernel(x, edge_index, W1, b1, Wmu, bmu, Wls, bls):
    n, d = x.shape
    e = edge_index.shape[1]
    ept = e // NW
    assert ept * NW == e and ept % 16 == 0

    np_ = -(-(n + 1) // 128) * 128

    nch = -(-ept // CH)
    ept_pad = nch * CH
    srcs = jnp.pad(edge_index[0].reshape(NW, ept),
                   ((0, 0), (0, ept_pad - ept))).reshape(NW, nch, CH)
    dpad = jnp.broadcast_to(
        n + jnp.arange(ept_pad - ept, dtype=edge_index.dtype) % (np_ - n),
        (NW, ept_pad - ept))
    dsts = jnp.concatenate(
        [edge_index[1].reshape(NW, ept), dpad], axis=1).reshape(NW, nch, CH)
    dstf = edge_index[1].reshape(NW, ept)
    zeros_nd = jnp.zeros((np_, d), jnp.float32)
    zeros_n = jnp.zeros((np_,), jnp.float32)

    cnt = _make_deg(np_, e)(dstf, zeros_n)
    dinv, y1 = _pre(cnt.T, x, W1)
    p = _make_prop(n, np_, d, ept_pad * NW)(y1, srcs, dsts, zeros_nd)
    y2 = _mid(p, y1, dinv, b1)
    q = _make_prop(n, np_, d, ept_pad * NW)(y2, srcs, dsts, zeros_nd)
    return _final(q, y2, dinv, Wmu, bmu, Wls, bls)


# speedup vs baseline: 1.4800x; 1.4800x over previous
import functools

import jax
import jax.numpy as jnp
from jax import lax
from jax.experimental import pallas as pl
from jax.experimental.pallas import tpu as pltpu
from jax.experimental.pallas import tpu_sc as plsc

NC = 2
NS = 16
NW = NC * NS
CH = 100


@functools.cache
def _make_prop(n: int, npad: int, d: int, e: int):
    ept = e // NW
    nch = ept // CH
    assert ept * NW == e and nch * CH == ept
    rpt = npad // NS
    assert rpt * NS == npad and rpt % 8 == 0
    mesh = plsc.VectorSubcoreMesh(core_axis_name="c", subcore_axis_name="s")

    @functools.partial(
        pl.kernel,
        out_type=jax.ShapeDtypeStruct((NC, npad, d), jnp.float32),
        mesh=mesh,
        scratch_types=[
            pltpu.VMEM((nch, CH), jnp.int32),
            pltpu.VMEM((nch, CH), jnp.int32),
            pltpu.VMEM((CH, d), jnp.float32),
            pltpu.VMEM_SHARED((npad, d), jnp.float32),
            pltpu.SemaphoreType.DMA,
        ],
    )
    def prop(y_hbm, srcs_hbm, dsts_hbm, zeros_hbm, out_hbm,
             src_v, dst_v, rows_v, accum, sem):
        c = lax.axis_index("c")
        s = lax.axis_index("s")
        wid = c * NS + s
        pltpu.sync_copy(srcs_hbm.at[wid], src_v)
        pltpu.sync_copy(dsts_hbm.at[wid], dst_v)
        r0 = s * rpt
        pltpu.sync_copy(zeros_hbm.at[pl.ds(r0, rpt)], accum.at[pl.ds(r0, rpt)])
        plsc.subcore_barrier()

        def body(j, carry):
            pltpu.async_copy(y_hbm.at[src_v.at[j]], rows_v, sem).wait()
            pltpu.sync_copy(rows_v, accum.at[dst_v.at[j]], add=True)
            return carry

        lax.fori_loop(0, nch, body, 0)
        plsc.subcore_barrier()
        pltpu.sync_copy(accum.at[pl.ds(r0, rpt)], out_hbm.at[c, pl.ds(r0, rpt)])

    return prop


@functools.cache
def _make_deg(n: int, e: int):
    ept = e // NW
    assert ept * NW == e and ept % 16 == 0
    mesh = plsc.VectorSubcoreMesh(core_axis_name="c", subcore_axis_name="s")

    @functools.partial(
        pl.kernel,
        out_type=jax.ShapeDtypeStruct((NW, n), jnp.float32),
        mesh=mesh,
        scratch_types=[
            pltpu.VMEM((ept,), jnp.int32),
            pltpu.VMEM((n,), jnp.float32),
        ],
        compiler_params=pltpu.CompilerParams(needs_layout_passes=False),
    )
    def deg(dsts_hbm, zeros_hbm, out_hbm, dst_v, cnt_v):
        c = lax.axis_index("c")
        s = lax.axis_index("s")
        wid = c * NS + s
        pltpu.sync_copy(dsts_hbm.at[wid], dst_v)
        pltpu.sync_copy(zeros_hbm, cnt_v)
        ones16 = jnp.ones((16,), jnp.float32)

        def body(i, carry):
            idx = dst_v[pl.ds(i * 16, 16)]
            plsc.addupdate_scatter(cnt_v, [idx], ones16)
            return carry

        lax.fori_loop(0, ept // 16, body, 0)
        pltpu.sync_copy(cnt_v, out_hbm.at[wid])

    return deg


def _blk(n):
    return next(b for b in (2048, 2000, 1280, 1000, 640, 512, 400, 200, 8)
                if n % b == 0)


def _pre_body(cnt_ref, x_ref, w_ref, dinv_ref, y_ref):
    deg = 1.0 + jnp.sum(cnt_ref[...], axis=1)
    dinv = lax.rsqrt(deg)[:, None]
    dinv_ref[...] = dinv
    y_ref[...] = jnp.dot(x_ref[...], w_ref[...],
                         preferred_element_type=jnp.float32) * dinv


def _pre(cnt_t, x, w):
    n, din = x.shape
    d = w.shape[1]
    blk = _blk(n)
    return pl.pallas_call(
        _pre_body,
        grid=(n // blk,),
        in_specs=[pl.BlockSpec((blk, NW), lambda i: (i, 0)),
                  pl.BlockSpec((blk, din), lambda i: (i, 0)),
                  pl.BlockSpec((din, d), lambda i: (0, 0))],
        out_specs=[pl.BlockSpec((blk, 1), lambda i: (i, 0)),
                   pl.BlockSpec((blk, d), lambda i: (i, 0))],
        out_shape=[jax.ShapeDtypeStruct((n, 1), jnp.float32),
                   jax.ShapeDtypeStruct((n, d), jnp.float32)],
    )(cnt_t, x, w)


def _mid_body(p_ref, y1_ref, dinv_ref, b1_ref, y2_ref):
    dinv = dinv_ref[...]
    pre = dinv * (p_ref[0] + p_ref[1] + y1_ref[...]) + b1_ref[...][None, :]
    y2_ref[...] = dinv * jnp.maximum(pre, 0.0)


def _mid(p, y1, dinv, b1):
    n, d = y1.shape
    blk = _blk(n)
    return pl.pallas_call(
        _mid_body,
        grid=(n // blk,),
        in_specs=[pl.BlockSpec((NC, blk, d), lambda i: (0, i, 0)),
                  pl.BlockSpec((blk, d), lambda i: (i, 0)),
                  pl.BlockSpec((blk, 1), lambda i: (i, 0)),
                  pl.BlockSpec((d,), lambda i: (0,))],
        out_specs=pl.BlockSpec((blk, d), lambda i: (i, 0)),
        out_shape=jax.ShapeDtypeStruct((n, d), jnp.float32),
    )(p, y1, dinv, b1)


def _final_body(q_ref, y2_ref, dinv_ref, wmu_ref, bmu_ref, wls_ref, bls_ref,
                mu_ref, ls_ref):
    g = dinv_ref[...] * (q_ref[0] + q_ref[1] + y2_ref[...])
    mu_ref[...] = (jnp.dot(g, wmu_ref[...], preferred_element_type=jnp.float32)
                   + bmu_ref[...][None, :])
    ls_ref[...] = (jnp.dot(g, wls_ref[...], preferred_element_type=jnp.float32)
                   + bls_ref[...][None, :])


def _final(q, y2, dinv, wmu, bmu, wls, bls):
    n, d = y2.shape
    dout = wmu.shape[1]
    blk = _blk(n)
    return pl.pallas_call(
        _final_body,
        grid=(n // blk,),
        in_specs=[pl.BlockSpec((NC, blk, d), lambda i: (0, i, 0)),
                  pl.BlockSpec((blk, d), lambda i: (i, 0)),
                  pl.BlockSpec((blk, 1), lambda i: (i, 0)),
                  pl.BlockSpec((d, dout), lambda i: (0, 0)),
                  pl.BlockSpec((dout,), lambda i: (0,)),
                  pl.BlockSpec((d, dout), lambda i: (0, 0)),
                  pl.BlockSpec((dout,), lambda i: (0,))],
        out_specs=[pl.BlockSpec((blk, dout), lambda i: (i, 0)),
                   pl.BlockSpec((blk, dout), lambda i: (i, 0))],
        out_shape=[jax.ShapeDtypeStruct((n, dout), jnp.float32),
                   jax.ShapeDtypeStruct((n, dout), jnp.float32)],
    )(q, y2, dinv, wmu, bmu, wls, bls)


def kernel(x, edge_index, W1, b1, Wmu, bmu, Wls, bls):
    n, d = x.shape
    e = edge_index.shape[1]
    ept = e // NW
    nch = ept // CH
    assert ept * NW == e and nch * CH == ept and ept % 16 == 0

    np_ = -(-n // 128) * 128

    srcs = edge_index[0].reshape(NW, nch, CH)
    dsts = edge_index[1].reshape(NW, nch, CH)
    dstf = edge_index[1].reshape(NW, ept)
    zeros_nd = jnp.zeros((np_, d), jnp.float32)
    zeros_n = jnp.zeros((np_,), jnp.float32)

    cnt = _make_deg(np_, e)(dstf, zeros_n)
    dinv, y1 = _pre(cnt.T, x, W1)
    p = _make_prop(n, np_, d, e)(y1, srcs, dsts, zeros_nd)
    y2 = _mid(p, y1, dinv, b1)
    q = _make_prop(n, np_, d, e)(y2, srcs, dsts, zeros_nd)
    return _final(q, y2, dinv, Wmu, bmu, Wls, bls)
